# baseline (speedup 1.0000x reference)
import functools

import jax
import jax.numpy as jnp
from jax import lax
from jax.experimental import pallas as pl
from jax.experimental.pallas import tpu as pltpu
from jax.experimental.pallas import tpu_sc as plsc

_B = 32
_C = 256
_H = 56
_LANES = 16
_NW = 32
_GROUPS = _C // _LANES
_ROWW = 7 * 2 * 8 * 128
_R = 2
_NCHUNKS = _H // _R
_WORDS = _R * _ROWW

_mesh = plsc.VectorSubcoreMesh(core_axis_name="c", subcore_axis_name="s")


@functools.partial(
    pl.kernel,
    out_type=jax.ShapeDtypeStruct((_B * _H * _ROWW,), jnp.float32),
    mesh=_mesh,
    scratch_types=[
        pltpu.VMEM((_C,), jnp.float32),
        pltpu.VMEM((_C,), jnp.int32),
        pltpu.VMEM((_C,), jnp.int32),
        pltpu.VMEM((_WORDS,), jnp.float32),
        pltpu.VMEM((_WORDS,), jnp.float32),
        pltpu.VMEM((_WORDS,), jnp.float32),
        pltpu.VMEM((_WORDS,), jnp.float32),
        pltpu.SemaphoreType.DMA,
        pltpu.SemaphoreType.DMA,
        pltpu.SemaphoreType.DMA,
        pltpu.SemaphoreType.DMA,
    ],
    compiler_params=pltpu.CompilerParams(
        use_tc_tiling_on_sc=False,
        needs_layout_passes=False,
    ),
)
def _channel_perm(in_hbm, idxs_hbm, out_hbm, idxs_v, sel_v, soff_v,
                  ibuf0, ibuf1, obuf0, obuf1, is0, is1, os0, os1):
    wid = lax.axis_index("c") * 16 + lax.axis_index("s")
    base = wid * (_H * _ROWW)

    ibufs = (ibuf0, ibuf1)
    obufs = (obuf0, obuf1)
    isems = (is0, is1)
    osems = (os0, os1)

    def in_slice(c):
        return in_hbm.at[pl.ds(base + c * _WORDS, _WORDS)]

    def out_slice(c):
        return out_hbm.at[pl.ds(base + c * _WORDS, _WORDS)]

    pltpu.async_copy(in_slice(0), ibufs[0], isems[0])
    pltpu.async_copy(in_slice(1), ibufs[1], isems[1])

    pltpu.sync_copy(idxs_hbm, idxs_v)

    for j in range(_GROUPS):
        sel_v[pl.ds(j * _LANES, _LANES)] = jnp.full(
            (_LANES,), _C - 1, jnp.int32)

    lane = lax.iota(jnp.int32, _LANES)

    def scan_body(i, cnt):
        v = idxs_v[pl.ds(i * _LANES, _LANES)]
        m = v != 0.0
        mi = jnp.where(m, jnp.int32(1), jnp.int32(0))
        csum = plsc.cumsum(mi)
        ranks = csum - 1 + cnt
        plsc.store_scatter(sel_v, [ranks], lane + i * _LANES, mask=m)
        return cnt + csum[15]

    count = lax.fori_loop(0, _GROUPS, scan_body, jnp.int32(0))
    countv = jnp.full((_LANES,), count, jnp.int32)
    nanv = jnp.full((_LANES,), jnp.nan, jnp.float32)

    def soff_body(j, carry):
        s16 = sel_v[pl.ds(j * _LANES, _LANES)]
        soff_v[pl.ds(j * _LANES, _LANES)] = (
            s16 + lax.shift_right_logical(s16, 7) * 896)
        return carry

    lax.fori_loop(0, _GROUPS, soff_body, jnp.int32(0))

    soffs = [soff_v[pl.ds(j * _LANES, _LANES)] for j in range(_GROUPS)]

    def compute_chunk(ib, ob):
        for r in range(_R):
            rbase = r * _ROWW

            @plsc.parallel_loop(0, 7 * 8, unroll=2)
            def spot_body(s):
                pw = rbase + s * 128 + lax.shift_right_logical(s, 3) * 1024
                for j in range(_GROUPS):
                    vals = plsc.load_gather(ib, [soffs[j] + pw])
                    koff = j * _LANES + (j // 8) * 896
                    ob[pl.ds(pw + koff, _LANES)] = vals

    def nan_fix_chunk(ob):
        for r in range(_R):
            rbase = r * _ROWW

            @plsc.parallel_loop(0, 7 * 8, unroll=1)
            def spot_body(s):
                pw = rbase + s * 128 + lax.shift_right_logical(s, 3) * 1024
                for j in range(_GROUPS):
                    @pl.when(count < (j + 1) * _LANES)
                    def _():
                        koff = j * _LANES + (j // 8) * 896
                        cur = ob[pl.ds(pw + koff, _LANES)]
                        k16 = lane + j * _LANES
                        ob[pl.ds(pw + koff, _LANES)] = jnp.where(
                            k16 < countv, cur, nanv)

    @pl.when(count == _C)
    def _identity_copy():
        img = _H * _ROWW
        cp = pltpu.async_copy(in_hbm.at[pl.ds(base, img)],
                              out_hbm.at[pl.ds(base, img)], os0)
        pltpu.make_async_copy(in_slice(0), ibufs[0], isems[0]).wait()
        pltpu.make_async_copy(in_slice(1), ibufs[1], isems[1]).wait()
        cp.wait()

    @pl.when(count < _C)
    def _general_path():
        def pair_body(i, carry):
            for b in range(2):
                c = 2 * i + b
                pltpu.make_async_copy(in_slice(c), ibufs[b], isems[b]).wait()

                @pl.when(c >= 2)
                def _():
                    pltpu.make_async_copy(
                        obufs[b], out_slice(c), osems[b]).wait()

                compute_chunk(ibufs[b], obufs[b])
                nan_fix_chunk(obufs[b])
                pltpu.async_copy(obufs[b], out_slice(c), osems[b])

                @pl.when(c + 2 < _NCHUNKS)
                def _():
                    pltpu.async_copy(in_slice(c + 2), ibufs[b], isems[b])
            return carry

        lax.fori_loop(0, _NCHUNKS // 2, pair_body, jnp.int32(0))

        pltpu.make_async_copy(obufs[0], out_slice(0), osems[0]).wait()
        pltpu.make_async_copy(obufs[1], out_slice(1), osems[1]).wait()


def kernel(input_tensor, indexes):
    b, c, h, w = input_tensor.shape
    six = input_tensor.reshape(b, 2, 128, h, 7, 8).transpose(0, 3, 4, 1, 5, 2)
    flat = six.reshape(-1)
    out = _channel_perm(flat, indexes)
    out6 = out.reshape(b, h, 7, 2, 8, 128)
    return out6.transpose(0, 3, 5, 1, 2, 4).reshape(b, c, h, w)


# speedup vs baseline: 33.6673x; 33.6673x over previous
import functools

import jax
import jax.numpy as jnp
from jax import lax
from jax.experimental import pallas as pl
from jax.experimental.pallas import tpu as pltpu
from jax.experimental.pallas import tpu_sc as plsc

_B = 32
_C = 256
_H = 56
_LANES = 16
_NW = 32
_GROUPS = _C // _LANES
_ROWW = 7 * 2 * 8 * 128
_R = 2
_NCHUNKS = _H // _R
_WORDS = _R * _ROWW

_mesh = plsc.VectorSubcoreMesh(core_axis_name="c", subcore_axis_name="s")


@functools.partial(
    pl.kernel,
    out_type=jax.ShapeDtypeStruct((_B * _H * _ROWW,), jnp.float32),
    mesh=_mesh,
    scratch_types=[
        pltpu.VMEM((_C,), jnp.float32),
        pltpu.VMEM((_C,), jnp.int32),
        pltpu.VMEM((_C,), jnp.int32),
        pltpu.VMEM((_WORDS,), jnp.float32),
        pltpu.VMEM((_WORDS,), jnp.float32),
        pltpu.VMEM((_WORDS,), jnp.float32),
        pltpu.VMEM((_WORDS,), jnp.float32),
        pltpu.SemaphoreType.DMA,
        pltpu.SemaphoreType.DMA,
        pltpu.SemaphoreType.DMA,
        pltpu.SemaphoreType.DMA,
    ],
    compiler_params=pltpu.CompilerParams(
        use_tc_tiling_on_sc=False,
        needs_layout_passes=False,
    ),
)
def _channel_perm(in_hbm, idxs_hbm, out_hbm, idxs_v, sel_v, soff_v,
                  ibuf0, ibuf1, obuf0, obuf1, is0, is1, os0, os1):
    wid = lax.axis_index("c") * 16 + lax.axis_index("s")
    base = wid * (_H * _ROWW)

    ibufs = (ibuf0, ibuf1)
    obufs = (obuf0, obuf1)
    isems = (is0, is1)
    osems = (os0, os1)

    def in_slice(c):
        return in_hbm.at[pl.ds(base + c * _WORDS, _WORDS)]

    def out_slice(c):
        return out_hbm.at[pl.ds(base + c * _WORDS, _WORDS)]

    pltpu.async_copy(in_slice(0), ibufs[0], isems[0])
    pltpu.async_copy(in_slice(1), ibufs[1], isems[1])

    pltpu.sync_copy(idxs_hbm, idxs_v)

    for j in range(_GROUPS):
        sel_v[pl.ds(j * _LANES, _LANES)] = jnp.full(
            (_LANES,), _C - 1, jnp.int32)

    lane = lax.iota(jnp.int32, _LANES)

    def scan_body(i, cnt):
        v = idxs_v[pl.ds(i * _LANES, _LANES)]
        m = v != 0.0
        mi = jnp.where(m, jnp.int32(1), jnp.int32(0))
        csum = plsc.cumsum(mi)
        ranks = csum - 1 + cnt
        plsc.store_scatter(sel_v, [ranks], lane + i * _LANES, mask=m)
        return cnt + csum[15]

    count = lax.fori_loop(0, _GROUPS, scan_body, jnp.int32(0))
    countv = jnp.full((_LANES,), count, jnp.int32)
    nanv = jnp.full((_LANES,), jnp.nan, jnp.float32)

    def soff_body(j, carry):
        s16 = sel_v[pl.ds(j * _LANES, _LANES)]
        soff_v[pl.ds(j * _LANES, _LANES)] = (
            s16 + lax.shift_right_logical(s16, 7) * 896)
        return carry

    lax.fori_loop(0, _GROUPS, soff_body, jnp.int32(0))

    soffs = [soff_v[pl.ds(j * _LANES, _LANES)] for j in range(_GROUPS)]

    def compute_chunk(ib, ob):
        for r in range(_R):
            rbase = r * _ROWW

            @plsc.parallel_loop(0, 7 * 8, unroll=2)
            def spot_body(s):
                pw = rbase + s * 128 + lax.shift_right_logical(s, 3) * 1024
                for j in range(_GROUPS):
                    vals = plsc.load_gather(ib, [soffs[j] + pw])
                    koff = j * _LANES + (j // 8) * 896
                    ob[pl.ds(pw + koff, _LANES)] = vals

    def nan_fix_chunk(ob):
        for r in range(_R):
            rbase = r * _ROWW

            @plsc.parallel_loop(0, 7 * 8, unroll=1)
            def spot_body(s):
                pw = rbase + s * 128 + lax.shift_right_logical(s, 3) * 1024
                for j in range(_GROUPS):
                    @pl.when(count < (j + 1) * _LANES)
                    def _():
                        koff = j * _LANES + (j // 8) * 896
                        cur = ob[pl.ds(pw + koff, _LANES)]
                        k16 = lane + j * _LANES
                        ob[pl.ds(pw + koff, _LANES)] = jnp.where(
                            k16 < countv, cur, nanv)

    @pl.when(count == _C)
    def _identity_copy():
        slots = (ibuf0, ibuf1, obuf0, obuf1)
        sems = (is0, is1, os0, os1)

        def ring_body(i, carry):
            for b in range(4):
                c = 4 * i + b
                pltpu.make_async_copy(in_slice(c), slots[b], sems[b]).wait()
                pltpu.async_copy(slots[b], out_slice(c), sems[b])
                b2 = (b + 2) % 4
                can_prefetch = c + 2 < _NCHUNKS

                @pl.when(jnp.logical_and(can_prefetch, c >= 2))
                def _():
                    pltpu.make_async_copy(
                        slots[b2], out_slice(c - 2), sems[b2]).wait()

                @pl.when(can_prefetch)
                def _():
                    pltpu.async_copy(in_slice(c + 2), slots[b2], sems[b2])
            return carry

        lax.fori_loop(0, _NCHUNKS // 4, ring_body, jnp.int32(0))

        for b in range(4):
            pltpu.make_async_copy(
                slots[b], out_slice(_NCHUNKS - 4 + b), sems[b]).wait()

    @pl.when(count < _C)
    def _general_path():
        def pair_body(i, carry):
            for b in range(2):
                c = 2 * i + b
                pltpu.make_async_copy(in_slice(c), ibufs[b], isems[b]).wait()

                @pl.when(c >= 2)
                def _():
                    pltpu.make_async_copy(
                        obufs[b], out_slice(c), osems[b]).wait()

                compute_chunk(ibufs[b], obufs[b])
                nan_fix_chunk(obufs[b])
                pltpu.async_copy(obufs[b], out_slice(c), osems[b])

                @pl.when(c + 2 < _NCHUNKS)
                def _():
                    pltpu.async_copy(in_slice(c + 2), ibufs[b], isems[b])
            return carry

        lax.fori_loop(0, _NCHUNKS // 2, pair_body, jnp.int32(0))

        pltpu.make_async_copy(obufs[0], out_slice(0), osems[0]).wait()
        pltpu.make_async_copy(obufs[1], out_slice(1), osems[1]).wait()


def kernel(input_tensor, indexes):
    b, c, h, w = input_tensor.shape
    six = input_tensor.reshape(b, 2, 128, h, 7, 8).transpose(0, 3, 4, 1, 5, 2)
    flat = six.reshape(-1)
    out = _channel_perm(flat, indexes)
    out6 = out.reshape(b, h, 7, 2, 8, 128)
    return out6.transpose(0, 3, 5, 1, 2, 4).reshape(b, c, h, w)


# speedup vs baseline: 33.7238x; 1.0017x over previous
import functools

import jax
import jax.numpy as jnp
from jax import lax
from jax.experimental import pallas as pl
from jax.experimental.pallas import tpu as pltpu
from jax.experimental.pallas import tpu_sc as plsc

_B = 32
_C = 256
_H = 56
_LANES = 16
_GROUPS = _C // _LANES
_ROWW = 7 * 2 * 8 * 128
_R = 2
_NCHUNKS = _H // _R
_WORDS = _R * _ROWW

_mesh = plsc.VectorSubcoreMesh(core_axis_name="c", subcore_axis_name="s")


@functools.partial(
    pl.kernel,
    out_type=jax.ShapeDtypeStruct((_B * _H * _ROWW,), jnp.float32),
    mesh=_mesh,
    scratch_types=[
        pltpu.VMEM((_C,), jnp.float32),
        pltpu.VMEM((_C,), jnp.int32),
        pltpu.VMEM((_C,), jnp.int32),
        pltpu.VMEM((_WORDS,), jnp.float32),
        pltpu.VMEM((_WORDS,), jnp.float32),
        pltpu.VMEM((_WORDS,), jnp.float32),
        pltpu.VMEM((_WORDS,), jnp.float32),
        pltpu.SemaphoreType.DMA,
        pltpu.SemaphoreType.DMA,
        pltpu.SemaphoreType.DMA,
        pltpu.SemaphoreType.DMA,
    ],
    compiler_params=pltpu.CompilerParams(
        use_tc_tiling_on_sc=False,
        needs_layout_passes=False,
    ),
)
def _channel_perm(in_hbm, idxs_hbm, out_hbm, idxs_v, sel_v, soff_v,
                  ibuf0, ibuf1, obuf0, obuf1, is0, is1, os0, os1):
    wid = lax.axis_index("c") * 16 + lax.axis_index("s")
    base = wid * (_H * _ROWW)

    ibufs = (ibuf0, ibuf1)
    obufs = (obuf0, obuf1)
    isems = (is0, is1)
    osems = (os0, os1)

    def in_slice(c):
        return in_hbm.at[pl.ds(base + c * _WORDS, _WORDS)]

    def out_slice(c):
        return out_hbm.at[pl.ds(base + c * _WORDS, _WORDS)]

    pltpu.async_copy(in_slice(0), ibufs[0], isems[0])
    pltpu.async_copy(in_slice(1), ibufs[1], isems[1])

    pltpu.sync_copy(idxs_hbm, idxs_v)

    for j in range(_GROUPS):
        sel_v[pl.ds(j * _LANES, _LANES)] = jnp.full(
            (_LANES,), _C - 1, jnp.int32)

    lane = lax.iota(jnp.int32, _LANES)

    def scan_body(i, cnt):
        v = idxs_v[pl.ds(i * _LANES, _LANES)]
        m = v != 0.0
        mi = jnp.where(m, jnp.int32(1), jnp.int32(0))
        csum = plsc.cumsum(mi)
        ranks = csum - 1 + cnt
        plsc.store_scatter(sel_v, [ranks], lane + i * _LANES, mask=m)
        return cnt + csum[15]

    count = lax.fori_loop(0, _GROUPS, scan_body, jnp.int32(0))
    countv = jnp.full((_LANES,), count, jnp.int32)
    nanv = jnp.full((_LANES,), jnp.nan, jnp.float32)

    def soff_body(j, carry):
        s16 = sel_v[pl.ds(j * _LANES, _LANES)]
        soff_v[pl.ds(j * _LANES, _LANES)] = (
            s16 + lax.shift_right_logical(s16, 7) * 896)
        return carry

    lax.fori_loop(0, _GROUPS, soff_body, jnp.int32(0))

    soffs = [soff_v[pl.ds(j * _LANES, _LANES)] for j in range(_GROUPS)]

    def compute_chunk(ib, ob):
        for r in range(_R):
            rbase = r * _ROWW

            @plsc.parallel_loop(0, 7 * 8, unroll=2)
            def spot_body(s):
                pw = rbase + s * 128 + lax.shift_right_logical(s, 3) * 1024
                for j in range(_GROUPS):
                    vals = plsc.load_gather(ib, [soffs[j] + pw])
                    koff = j * _LANES + (j // 8) * 896
                    ob[pl.ds(pw + koff, _LANES)] = vals

    def nan_fix_chunk(ob):
        for r in range(_R):
            rbase = r * _ROWW

            @plsc.parallel_loop(0, 7 * 8, unroll=1)
            def spot_body(s):
                pw = rbase + s * 128 + lax.shift_right_logical(s, 3) * 1024
                for j in range(_GROUPS):
                    @pl.when(count < (j + 1) * _LANES)
                    def _():
                        koff = j * _LANES + (j // 8) * 896
                        cur = ob[pl.ds(pw + koff, _LANES)]
                        k16 = lane + j * _LANES
                        ob[pl.ds(pw + koff, _LANES)] = jnp.where(
                            k16 < countv, cur, nanv)

    @pl.when(count == _C)
    def _identity_copy():
        slots = (ibuf0, ibuf1, obuf0, obuf1)
        sems = (is0, is1, os0, os1)

        def ring_body(i, carry):
            for b in range(4):
                c = 4 * i + b
                pltpu.make_async_copy(in_slice(c), slots[b], sems[b]).wait()
                pltpu.async_copy(slots[b], out_slice(c), sems[b])
                b2 = (b + 2) % 4
                can_prefetch = c + 2 < _NCHUNKS

                @pl.when(jnp.logical_and(can_prefetch, c >= 2))
                def _():
                    pltpu.make_async_copy(
                        slots[b2], out_slice(c - 2), sems[b2]).wait()

                @pl.when(can_prefetch)
                def _():
                    pltpu.async_copy(in_slice(c + 2), slots[b2], sems[b2])
            return carry

        lax.fori_loop(0, _NCHUNKS // 4, ring_body, jnp.int32(0))

        for b in range(4):
            pltpu.make_async_copy(
                slots[b], out_slice(_NCHUNKS - 4 + b), sems[b]).wait()

    @pl.when(count < _C)
    def _general_path():
        def pair_body(i, carry):
            for b in range(2):
                c = 2 * i + b
                pltpu.make_async_copy(in_slice(c), ibufs[b], isems[b]).wait()

                @pl.when(c >= 2)
                def _():
                    pltpu.make_async_copy(
                        obufs[b], out_slice(c), osems[b]).wait()

                compute_chunk(ibufs[b], obufs[b])
                nan_fix_chunk(obufs[b])
                pltpu.async_copy(obufs[b], out_slice(c), osems[b])

                @pl.when(c + 2 < _NCHUNKS)
                def _():
                    pltpu.async_copy(in_slice(c + 2), ibufs[b], isems[b])
            return carry

        lax.fori_loop(0, _NCHUNKS // 2, pair_body, jnp.int32(0))

        pltpu.make_async_copy(obufs[0], out_slice(0), osems[0]).wait()
        pltpu.make_async_copy(obufs[1], out_slice(1), osems[1]).wait()


def kernel(input_tensor, indexes):
    b, c, h, w = input_tensor.shape
    six = input_tensor.reshape(b, 2, 128, h, 7, 8).transpose(0, 3, 4, 1, 5, 2)
    flat = six.reshape(-1)
    out = _channel_perm(flat, indexes)
    out6 = out.reshape(b, h, 7, 2, 8, 128)
    return out6.transpose(0, 3, 5, 1, 2, 4).reshape(b, c, h, w)
